# baseline (speedup 1.0000x reference)
import functools

import jax
import jax.numpy as jnp
import numpy as np
from jax.experimental import pallas as pl
from jax.experimental.pallas import tpu as pltpu

_LOSS_COEF = 1e-2
_EPS = float(np.finfo(float).eps)


def _cv_sq(v):
    n = v.shape[0]
    mu = jnp.mean(v)
    var = jnp.sum((v - mu) ** 2) / (n - 1)
    return var / (mu * mu + 1e-10)


def _gating_kernel(x_ref, wg_ref, gates_ref, loss_ref):
    x = x_ref[...]
    wg = wg_ref[...]
    logits = jax.lax.dot_general(
        x, wg, (((1,), (0,)), ((), ())), preferred_element_type=jnp.float32
    )
    ne = logits.shape[1]
    cols = jax.lax.broadcasted_iota(jnp.int32, logits.shape, 1)
    m1 = jnp.max(logits, axis=1, keepdims=True)
    i1 = jnp.min(jnp.where(logits == m1, cols, ne), axis=1, keepdims=True)
    masked = jnp.where(cols == i1, -jnp.inf, logits)
    m2 = jnp.max(masked, axis=1, keepdims=True)
    i2 = jnp.min(jnp.where(masked == m2, cols, ne), axis=1, keepdims=True)
    e2 = jnp.exp(m2 - m1)
    denom = 1.0 + e2
    g1 = 1.0 / denom
    g2 = e2 / denom
    gates = (
        jnp.where(cols == i1, g1, 0.0) + jnp.where(cols == i2, g2, 0.0)
    )
    gates_ref[...] = gates
    importance = jnp.sum(gates, axis=0)
    load = jnp.sum((gates > 0).astype(jnp.float32), axis=0)
    loss = (_cv_sq(importance) + _cv_sq(load)) * _LOSS_COEF
    loss_ref[...] = loss[None, None]


def _expert_kernel(
    x_ref, gates_ref, W1_ref, b1_ref, W2_ref, b2_ref, out_ref, oe_acc, *, nhb, tt, ne
):
    e = pl.program_id(0)
    hb = pl.program_id(1)
    t = pl.program_id(2)
    T = out_ref.shape[0] // tt

    xt = x_ref[pl.ds(t * T, T), :]
    h = jax.lax.dot_general(
        xt, W1_ref[0], (((1,), (0,)), ((), ())), preferred_element_type=jnp.float32
    )
    h = jnp.maximum(h + b1_ref[0], 0.0)
    partial = jax.lax.dot_general(
        h, W2_ref[0], (((1,), (0,)), ((), ())), preferred_element_type=jnp.float32
    )

    rows = pl.ds(t * T, T)

    @pl.when(hb == 0)
    def _():
        oe_acc[rows, :] = partial

    @pl.when(hb > 0)
    def _():
        oe_acc[rows, :] += partial

    @pl.when(hb == nhb - 1)
    def _():
        oe = oe_acc[rows, :] + b2_ref[0]
        g = gates_ref[rows, :]
        ecols = jax.lax.broadcasted_iota(jnp.int32, g.shape, 1)
        ge = jnp.sum(jnp.where(ecols == e, g, 0.0), axis=1, keepdims=True)
        contrib = jnp.where(ge > 0, jnp.exp(oe) * ge, 0.0)

        @pl.when(e == 0)
        def _():
            out_ref[rows, :] = contrib

        @pl.when(e > 0)
        def _():
            acc = out_ref[rows, :] + contrib

            @pl.when(e < ne - 1)
            def _():
                out_ref[rows, :] = acc

            @pl.when(e == ne - 1)
            def _():
                out_ref[rows, :] = jnp.log(
                    jnp.where(acc == 0.0, jnp.float32(_EPS), acc)
                )


def kernel(x, w_gate, W1, b1, W2, b2):
    B, D = x.shape
    ne = W1.shape[0]
    H = W1.shape[2]
    O = W2.shape[2]

    gates, loss2d = pl.pallas_call(
        _gating_kernel,
        out_shape=(
            jax.ShapeDtypeStruct((B, ne), jnp.float32),
            jax.ShapeDtypeStruct((1, 1), jnp.float32),
        ),
    )(x, w_gate)

    HB = min(1024, H)
    nhb = H // HB
    T = min(256, B)
    tt = B // T

    b1r = b1.reshape(ne, 1, H)
    b2r = b2.reshape(ne, 1, O)

    y = pl.pallas_call(
        functools.partial(_expert_kernel, nhb=nhb, tt=tt, ne=ne),
        grid=(ne, nhb, tt),
        in_specs=[
            pl.BlockSpec((B, D), lambda e, hb, t: (0, 0)),
            pl.BlockSpec((B, ne), lambda e, hb, t: (0, 0)),
            pl.BlockSpec((1, D, HB), lambda e, hb, t: (e, 0, hb)),
            pl.BlockSpec((1, 1, HB), lambda e, hb, t: (e, 0, hb)),
            pl.BlockSpec((1, HB, O), lambda e, hb, t: (e, hb, 0)),
            pl.BlockSpec((1, 1, O), lambda e, hb, t: (e, 0, 0)),
        ],
        out_specs=pl.BlockSpec((B, O), lambda e, hb, t: (0, 0)),
        out_shape=jax.ShapeDtypeStruct((B, O), jnp.float32),
        scratch_shapes=[pltpu.VMEM((B, O), jnp.float32)],
        compiler_params=pltpu.CompilerParams(
            dimension_semantics=("arbitrary", "arbitrary", "arbitrary"),
        ),
    )(x, gates, W1, b1r, W2, b2r)

    return y, loss2d[0, 0]


# speedup vs baseline: 1.1315x; 1.1315x over previous
import functools

import jax
import jax.numpy as jnp
import numpy as np
from jax import lax
from jax.experimental import pallas as pl
from jax.experimental.pallas import tpu as pltpu
from jax.experimental.pallas import tpu_sc as plsc

_LOSS_COEF = 1e-2
_EPS = float(np.finfo(float).eps)

_T = 256
_HB = 1024


def _cv_sq(v):
    n = v.shape[0]
    mu = jnp.mean(v)
    var = jnp.sum((v - mu) ** 2) / (n - 1)
    return var / (mu * mu + 1e-10)


def _routing_kernel(x_ref, wg_ref, loss_ref, g2d_ref, dest_ref, texp_ref,
                    nt_ref, *, maxt):
    x = x_ref[...]
    wg = wg_ref[...]
    B = x.shape[0]
    ne = wg.shape[1]
    logits = lax.dot_general(
        x, wg, (((1,), (0,)), ((), ())), preferred_element_type=jnp.float32
    )
    cols = lax.broadcasted_iota(jnp.int32, logits.shape, 1)
    m1 = jnp.max(logits, axis=1, keepdims=True)
    i1 = jnp.min(jnp.where(logits == m1, cols, ne), axis=1, keepdims=True)
    masked = jnp.where(cols == i1, -jnp.inf, logits)
    m2 = jnp.max(masked, axis=1, keepdims=True)
    i2 = jnp.min(jnp.where(masked == m2, cols, ne), axis=1, keepdims=True)
    e2 = jnp.exp(m2 - m1)
    denom = 1.0 + e2
    g1 = 1.0 / denom
    g2 = e2 / denom

    oh1 = (cols == i1).astype(jnp.float32)
    oh2 = (cols == i2).astype(jnp.float32)
    gates = oh1 * g1 + jnp.where(g2 > 0, oh2 * g2, 0.0)
    importance = jnp.sum(gates, axis=0)
    load = jnp.sum((gates > 0).astype(jnp.float32), axis=0)
    loss_ref[...] = ((_cv_sq(importance) + _cv_sq(load)) * _LOSS_COEF)[
        None, None
    ]

    gcols = lax.broadcasted_iota(jnp.int32, g2d_ref.shape, 1)
    g2d_ref[...] = jnp.where(
        gcols == 0, g1, jnp.where(gcols == 1, g2, 0.0)
    )

    oh = jnp.concatenate([oh1, oh2], axis=0)
    c = oh
    step = 1
    while step < 2 * B:
        c = c + jnp.concatenate(
            [jnp.zeros((step, ne), jnp.float32), c[: 2 * B - step, :]], axis=0
        )
        step *= 2
    counts = c[2 * B - 1 : 2 * B, :]
    cnt_pad = jnp.ceil(counts / _T) * _T
    rl = lax.broadcasted_iota(jnp.int32, (ne, ne), 0)
    cl = lax.broadcasted_iota(jnp.int32, (ne, ne), 1)
    lower = (rl < cl).astype(jnp.float32)
    offs = lax.dot_general(
        cnt_pad, lower, (((1,), (0,)), ((), ())),
        preferred_element_type=jnp.float32,
    )
    ohs = jnp.concatenate([oh1, oh2], axis=0)
    dest = jnp.sum(ohs * (offs + c - 1.0), axis=1, keepdims=True)
    dest_ref[...] = dest.astype(jnp.int32)

    ends = offs + cnt_pad
    jt = lax.broadcasted_iota(jnp.int32, (maxt, 1), 0).astype(jnp.float32) * _T
    texp = jnp.sum((jt >= ends).astype(jnp.float32), axis=1, keepdims=True)
    texp_ref[...] = jnp.minimum(texp, float(ne - 1)).astype(jnp.int32)
    nt_ref[...] = (jnp.sum(cnt_pad) / _T).astype(jnp.int32)[None, None]


def _expert_kernel(texp_ref, nt_ref, xs_ref, W1_ref, b1_ref, W2_ref, b2_ref,
                   out_ref, oe_acc, *, nhb):
    hb = pl.program_id(0)
    j = pl.program_id(1)
    T = out_ref.shape[0]

    @pl.when(j < nt_ref[0])
    def _():
        h = lax.dot_general(
            xs_ref[...], W1_ref[0], (((1,), (0,)), ((), ())),
            preferred_element_type=jnp.float32,
        )
        h = jnp.maximum(h + b1_ref[0], 0.0)
        partial = lax.dot_general(
            h, W2_ref[0], (((1,), (0,)), ((), ())),
            preferred_element_type=jnp.float32,
        )
        rows = pl.ds(j * T, T)

        @pl.when(hb == 0)
        def _():
            oe_acc[rows, :] = partial

        @pl.when(hb > 0)
        def _():
            oe_acc[rows, :] += partial

        @pl.when(hb == nhb - 1)
        def _():
            out_ref[...] = jnp.exp(oe_acc[rows, :] + b2_ref[0])


def _finalize_kernel(c_ref, g2d_ref, y_ref):
    c0 = c_ref[0]
    c1 = c_ref[1]
    g1 = g2d_ref[:, 0:1]
    g2 = g2d_ref[:, 1:2]
    acc = jnp.where(g1 > 0, g1 * c0, 0.0) + jnp.where(g2 > 0, g2 * c1, 0.0)
    y_ref[...] = jnp.log(jnp.where(acc == 0.0, jnp.float32(_EPS), acc))


def _sc_build_src_tok(dest, buf_rows):
    M = dest.shape[0]
    B = M // 2
    info = plsc.get_sparse_core_info()

    @functools.partial(
        pl.kernel,
        out_type=jax.ShapeDtypeStruct((buf_rows,), jnp.int32),
        mesh=plsc.VectorSubcoreMesh(core_axis_name="c", subcore_axis_name="s"),
        scratch_types=[
            pltpu.VMEM((M,), jnp.int32),
            pltpu.VMEM((buf_rows,), jnp.int32),
        ],
        compiler_params=pltpu.CompilerParams(needs_layout_passes=False),
    )
    def k(dest_hbm, out_hbm, dest_v, tok_v):
        wid = lax.axis_index("s") * info.num_cores + lax.axis_index("c")

        @pl.when(wid == 0)
        def _():
            zeros = jnp.zeros((16,), jnp.int32)

            def init_body(i, _):
                tok_v[pl.ds(i * 16, 16)] = zeros
                return 0

            lax.fori_loop(0, buf_rows // 16, init_body, 0)
            pltpu.sync_copy(dest_hbm, dest_v)
            lane = lax.iota(jnp.int32, 16)

            def body(i, _):
                idx = dest_v[pl.ds(i * 16, 16)]
                tok = (lane + i * 16) & (B - 1)
                plsc.store_scatter(tok_v, [idx], tok)
                return 0

            lax.fori_loop(0, M // 16, body, 0)
            pltpu.sync_copy(tok_v, out_hbm)

    return k(dest)


def _sc_gather_rows(table, idx, n_chunks):
    M = idx.shape[0]
    D = table.shape[1]
    info = plsc.get_sparse_core_info()
    NW = info.num_cores * info.num_subcores
    per_w = M // NW
    ch = per_w // n_chunks

    @functools.partial(
        pl.kernel,
        out_type=jax.ShapeDtypeStruct((M, D), jnp.float32),
        mesh=plsc.VectorSubcoreMesh(core_axis_name="c", subcore_axis_name="s"),
        scratch_types=[
            pltpu.VMEM((ch,), jnp.int32),
            pltpu.VMEM((ch, D), jnp.float32),
            pltpu.SemaphoreType.DMA,
        ],
    )
    def k(table_hbm, idx_hbm, out_hbm, idx_v, rows_v, sem):
        wid = lax.axis_index("s") * info.num_cores + lax.axis_index("c")
        base = wid * per_w
        for c in range(n_chunks):
            off = base + c * ch
            pltpu.sync_copy(idx_hbm.at[pl.ds(off, ch)], idx_v)
            pltpu.async_copy(table_hbm.at[idx_v], rows_v, sem).wait()
            pltpu.sync_copy(rows_v, out_hbm.at[pl.ds(off, ch)])

    return k(table, idx)


def kernel(x, w_gate, W1, b1, W2, b2):
    B, D = x.shape
    ne = W1.shape[0]
    H = W1.shape[2]
    O = W2.shape[2]
    hbw = min(_HB, H)
    nhb = H // hbw
    maxt = -(-2 * B // _T) + ne - 1
    maxt += (-maxt) % 8
    buf = maxt * _T

    loss2d, g2d, dest2d, texp2d, nt2d = pl.pallas_call(
        functools.partial(_routing_kernel, maxt=maxt),
        out_shape=(
            jax.ShapeDtypeStruct((1, 1), jnp.float32),
            jax.ShapeDtypeStruct((B, 128), jnp.float32),
            jax.ShapeDtypeStruct((2 * B, 1), jnp.int32),
            jax.ShapeDtypeStruct((maxt, 1), jnp.int32),
            jax.ShapeDtypeStruct((1, 1), jnp.int32),
        ),
    )(x, w_gate)

    dest = dest2d.reshape(2 * B)
    src_tok = _sc_build_src_tok(dest, buf)
    xs = _sc_gather_rows(x, src_tok, 2)

    b1r = b1.reshape(ne, 1, H)
    b2r = b2.reshape(ne, 1, O)
    texp = texp2d.reshape(maxt)
    nt = nt2d.reshape(1)

    contrib = pl.pallas_call(
        functools.partial(_expert_kernel, nhb=nhb),
        grid_spec=pltpu.PrefetchScalarGridSpec(
            num_scalar_prefetch=2,
            grid=(nhb, maxt),
            in_specs=[
                pl.BlockSpec((_T, D), lambda hb, j, texp, nt: (j, 0)),
                pl.BlockSpec((1, D, hbw), lambda hb, j, texp, nt: (texp[j], 0, hb)),
                pl.BlockSpec((1, 1, hbw), lambda hb, j, texp, nt: (texp[j], 0, hb)),
                pl.BlockSpec((1, hbw, O), lambda hb, j, texp, nt: (texp[j], hb, 0)),
                pl.BlockSpec((1, 1, O), lambda hb, j, texp, nt: (texp[j], 0, 0)),
            ],
            out_specs=pl.BlockSpec((_T, O), lambda hb, j, texp, nt: (j, 0)),
            scratch_shapes=[pltpu.VMEM((buf, O), jnp.float32)],
        ),
        out_shape=jax.ShapeDtypeStruct((buf, O), jnp.float32),
        compiler_params=pltpu.CompilerParams(
            dimension_semantics=("arbitrary", "arbitrary"),
        ),
    )(texp, nt, xs, W1, b1r, W2, b2r)

    crows = _sc_gather_rows(contrib, dest, 2).reshape(2, B, O)

    y = pl.pallas_call(
        _finalize_kernel,
        grid=(B // _T,),
        in_specs=[
            pl.BlockSpec((2, _T, O), lambda t: (0, t, 0)),
            pl.BlockSpec((_T, 128), lambda t: (t, 0)),
        ],
        out_specs=pl.BlockSpec((_T, O), lambda t: (t, 0)),
        out_shape=jax.ShapeDtypeStruct((B, O), jnp.float32),
    )(crows, g2d)

    return y, loss2d[0, 0]


# speedup vs baseline: 1.1747x; 1.0381x over previous
import functools

import jax
import jax.numpy as jnp
import numpy as np
from jax import lax
from jax.experimental import pallas as pl
from jax.experimental.pallas import tpu as pltpu
from jax.experimental.pallas import tpu_sc as plsc

_LOSS_COEF = 1e-2
_EPS = float(np.finfo(float).eps)

_T = 256
_HB = 1024


def _cv_sq(v):
    n = v.shape[0]
    mu = jnp.mean(v)
    var = jnp.sum((v - mu) ** 2) / (n - 1)
    return var / (mu * mu + 1e-10)


def _routing_kernel(x_ref, wg_ref, loss_ref, g2d_ref, dest_ref, texp_ref,
                    nt_ref, *, maxt):
    x = x_ref[...]
    wg = wg_ref[...]
    B = x.shape[0]
    ne = wg.shape[1]
    logits = lax.dot_general(
        x, wg, (((1,), (0,)), ((), ())), preferred_element_type=jnp.float32
    )
    cols = lax.broadcasted_iota(jnp.int32, logits.shape, 1)
    m1 = jnp.max(logits, axis=1, keepdims=True)
    i1 = jnp.min(jnp.where(logits == m1, cols, ne), axis=1, keepdims=True)
    masked = jnp.where(cols == i1, -jnp.inf, logits)
    m2 = jnp.max(masked, axis=1, keepdims=True)
    i2 = jnp.min(jnp.where(masked == m2, cols, ne), axis=1, keepdims=True)
    e2 = jnp.exp(m2 - m1)
    denom = 1.0 + e2
    g1 = 1.0 / denom
    g2 = e2 / denom

    oh1 = (cols == i1).astype(jnp.float32)
    oh2 = (cols == i2).astype(jnp.float32)
    gates = oh1 * g1 + jnp.where(g2 > 0, oh2 * g2, 0.0)
    importance = jnp.sum(gates, axis=0)
    load = jnp.sum((gates > 0).astype(jnp.float32), axis=0)
    loss_ref[...] = ((_cv_sq(importance) + _cv_sq(load)) * _LOSS_COEF)[
        None, None
    ]

    gcols = lax.broadcasted_iota(jnp.int32, g2d_ref.shape, 1)
    g2d_ref[...] = jnp.where(
        gcols == 0, g1, jnp.where(gcols == 1, g2, 0.0)
    )

    oh = jnp.concatenate([oh1, oh2], axis=0)
    c = oh
    step = 1
    while step < 2 * B:
        c = c + jnp.concatenate(
            [jnp.zeros((step, ne), jnp.float32), c[: 2 * B - step, :]], axis=0
        )
        step *= 2
    counts = c[2 * B - 1 : 2 * B, :]
    cnt_pad = jnp.ceil(counts / _T) * _T
    rl = lax.broadcasted_iota(jnp.int32, (ne, ne), 0)
    cl = lax.broadcasted_iota(jnp.int32, (ne, ne), 1)
    lower = (rl < cl).astype(jnp.float32)
    offs = lax.dot_general(
        cnt_pad, lower, (((1,), (0,)), ((), ())),
        preferred_element_type=jnp.float32,
    )
    ohs = jnp.concatenate([oh1, oh2], axis=0)
    dest = jnp.sum(ohs * (offs + c - 1.0), axis=1, keepdims=True)
    dest_ref[...] = dest.astype(jnp.int32)

    ends = offs + cnt_pad
    jt = lax.broadcasted_iota(jnp.int32, (maxt, 1), 0).astype(jnp.float32) * _T
    texp = jnp.sum((jt >= ends).astype(jnp.float32), axis=1, keepdims=True)
    texp_ref[...] = jnp.minimum(texp, float(ne - 1)).astype(jnp.int32)
    nt_ref[...] = (jnp.sum(cnt_pad) / _T).astype(jnp.int32)[None, None]


def _expert_kernel(texp_ref, nt_ref, xs_ref, W1_ref, b1_ref, W2_ref, b2_ref,
                   out_ref, oe_acc, sem, *, nhb, tt):
    hb = pl.program_id(0)
    j = pl.program_id(1)
    T = xs_ref.shape[0]

    @pl.when(j < nt_ref[0])
    def _():
        rows = pl.ds(j * T, T)
        h = lax.dot_general(
            xs_ref[...], W1_ref[0], (((1,), (0,)), ((), ())),
            preferred_element_type=jnp.float32,
        )
        h = jnp.maximum(h + b1_ref[0], 0.0)
        partial = lax.dot_general(
            h, W2_ref[0], (((1,), (0,)), ((), ())),
            preferred_element_type=jnp.float32,
        )

        @pl.when(hb == 0)
        def _():
            oe_acc[rows, :] = partial

        @pl.when(hb > 0)
        def _():
            oe_acc[rows, :] += partial

        @pl.when(hb == nhb - 1)
        def _():
            oe_acc[rows, :] = jnp.exp(oe_acc[rows, :] + b2_ref[0])
            copy = pltpu.make_async_copy(
                oe_acc.at[rows, :], out_ref.at[rows, :], sem
            )
            copy.start()
            copy.wait()


def _finalize_kernel(c_ref, g2d_ref, y_ref):
    c0 = c_ref[0]
    c1 = c_ref[1]
    g1 = g2d_ref[:, 0:1]
    g2 = g2d_ref[:, 1:2]
    acc = jnp.where(g1 > 0, g1 * c0, 0.0) + jnp.where(g2 > 0, g2 * c1, 0.0)
    y_ref[...] = jnp.log(jnp.where(acc == 0.0, jnp.float32(_EPS), acc))


def _sc_dispatch(x, dest, buf_rows, n_chunks):
    M = dest.shape[0]
    B, D = x.shape
    info = plsc.get_sparse_core_info()
    NW = info.num_cores * info.num_subcores
    per_w = buf_rows // NW
    ch = per_w // n_chunks

    @functools.partial(
        pl.kernel,
        out_type=jax.ShapeDtypeStruct((buf_rows, D), jnp.float32),
        mesh=plsc.VectorSubcoreMesh(core_axis_name="c", subcore_axis_name="s"),
        scratch_types=[
            pltpu.VMEM((M,), jnp.int32),
            pltpu.VMEM((buf_rows,), jnp.int32),
            pltpu.VMEM_SHARED((buf_rows,), jnp.int32),
            pltpu.VMEM((ch,), jnp.int32),
            pltpu.VMEM((ch, D), jnp.float32),
            pltpu.SemaphoreType.DMA,
        ],
        compiler_params=pltpu.CompilerParams(needs_layout_passes=False),
    )
    def k(x_hbm, dest_hbm, out_hbm, dest_v, tok_v, tok_s, idx_v, rows_v, sem):
        cid = lax.axis_index("c")
        sid = lax.axis_index("s")
        wid = sid * info.num_cores + cid

        @pl.when(sid == 0)
        def _():
            zeros = jnp.zeros((16,), jnp.int32)

            def init_body(i, _):
                tok_v[pl.ds(i * 16, 16)] = zeros
                return 0

            lax.fori_loop(0, buf_rows // 16, init_body, 0)
            pltpu.sync_copy(dest_hbm, dest_v)
            lane = lax.iota(jnp.int32, 16)

            def body(i, _):
                idx = dest_v[pl.ds(i * 16, 16)]
                tok = (lane + i * 16) & (B - 1)
                plsc.store_scatter(tok_v, [idx], tok)
                return 0

            lax.fori_loop(0, M // 16, body, 0)
            pltpu.sync_copy(tok_v, tok_s)

        plsc.subcore_barrier()
        base = wid * per_w
        for c in range(n_chunks):
            off = base + c * ch
            pltpu.sync_copy(tok_s.at[pl.ds(off, ch)], idx_v)
            pltpu.async_copy(x_hbm.at[idx_v], rows_v, sem).wait()
            pltpu.sync_copy(rows_v, out_hbm.at[pl.ds(off, ch)])

    return k(x, dest)


def _sc_gather_rows(table, idx, n_chunks):
    M = idx.shape[0]
    D = table.shape[1]
    info = plsc.get_sparse_core_info()
    NW = info.num_cores * info.num_subcores
    per_w = M // NW
    ch = per_w // n_chunks

    @functools.partial(
        pl.kernel,
        out_type=jax.ShapeDtypeStruct((M, D), jnp.float32),
        mesh=plsc.VectorSubcoreMesh(core_axis_name="c", subcore_axis_name="s"),
        scratch_types=[
            pltpu.VMEM((ch,), jnp.int32),
            pltpu.VMEM((ch, D), jnp.float32),
            pltpu.SemaphoreType.DMA,
        ],
    )
    def k(table_hbm, idx_hbm, out_hbm, idx_v, rows_v, sem):
        wid = lax.axis_index("s") * info.num_cores + lax.axis_index("c")
        base = wid * per_w
        for c in range(n_chunks):
            off = base + c * ch
            pltpu.sync_copy(idx_hbm.at[pl.ds(off, ch)], idx_v)
            pltpu.async_copy(table_hbm.at[idx_v], rows_v, sem).wait()
            pltpu.sync_copy(rows_v, out_hbm.at[pl.ds(off, ch)])

    return k(table, idx)


def kernel(x, w_gate, W1, b1, W2, b2):
    B, D = x.shape
    ne = W1.shape[0]
    H = W1.shape[2]
    O = W2.shape[2]
    hbw = min(_HB, H)
    nhb = H // hbw
    maxt = -(-2 * B // _T) + ne - 1
    maxt += (-maxt) % 8
    buf = maxt * _T

    loss2d, g2d, dest2d, texp2d, nt2d = pl.pallas_call(
        functools.partial(_routing_kernel, maxt=maxt),
        out_shape=(
            jax.ShapeDtypeStruct((1, 1), jnp.float32),
            jax.ShapeDtypeStruct((B, 128), jnp.float32),
            jax.ShapeDtypeStruct((2 * B, 1), jnp.int32),
            jax.ShapeDtypeStruct((maxt, 1), jnp.int32),
            jax.ShapeDtypeStruct((1, 1), jnp.int32),
        ),
    )(x, w_gate)

    dest = dest2d.reshape(2 * B)
    xs = _sc_dispatch(x, dest, buf, 2)

    b1r = b1.reshape(ne, 1, H)
    b2r = b2.reshape(ne, 1, O)
    texp = texp2d.reshape(maxt)
    nt = nt2d.reshape(1)

    contrib = pl.pallas_call(
        functools.partial(_expert_kernel, nhb=nhb, tt=maxt),
        grid_spec=pltpu.PrefetchScalarGridSpec(
            num_scalar_prefetch=2,
            grid=(nhb, maxt),
            in_specs=[
                pl.BlockSpec((_T, D), lambda hb, j, texp, nt: (j, 0)),
                pl.BlockSpec((1, D, hbw), lambda hb, j, texp, nt: (texp[j], 0, hb)),
                pl.BlockSpec((1, 1, hbw), lambda hb, j, texp, nt: (texp[j], 0, hb)),
                pl.BlockSpec((1, hbw, O), lambda hb, j, texp, nt: (texp[j], hb, 0)),
                pl.BlockSpec((1, 1, O), lambda hb, j, texp, nt: (texp[j], 0, 0)),
            ],
            out_specs=pl.BlockSpec(memory_space=pl.ANY),
            scratch_shapes=[
                pltpu.VMEM((buf, O), jnp.float32),
                pltpu.SemaphoreType.DMA,
            ],
        ),
        out_shape=jax.ShapeDtypeStruct((buf, O), jnp.float32),
        compiler_params=pltpu.CompilerParams(
            dimension_semantics=("arbitrary", "arbitrary"),
            vmem_limit_bytes=100 * 1024 * 1024,
        ),
    )(texp, nt, xs, W1, b1r, W2, b2r)

    crows = _sc_gather_rows(contrib, dest, 2).reshape(2, B, O)

    y = pl.pallas_call(
        _finalize_kernel,
        grid=(B // _T,),
        in_specs=[
            pl.BlockSpec((2, _T, O), lambda t: (0, t, 0)),
            pl.BlockSpec((_T, 128), lambda t: (t, 0)),
        ],
        out_specs=pl.BlockSpec((_T, O), lambda t: (t, 0)),
        out_shape=jax.ShapeDtypeStruct((B, O), jnp.float32),
    )(crows, g2d)

    return y, loss2d[0, 0]


# speedup vs baseline: 1.1761x; 1.0012x over previous
import functools

import jax
import jax.numpy as jnp
import numpy as np
from jax import lax
from jax.experimental import pallas as pl
from jax.experimental.pallas import tpu as pltpu
from jax.experimental.pallas import tpu_sc as plsc

_LOSS_COEF = 1e-2
_EPS = float(np.finfo(float).eps)

_T = 256
_HB = 1024


def _cv_sq(v):
    n = v.shape[0]
    mu = jnp.mean(v)
    var = jnp.sum((v - mu) ** 2) / (n - 1)
    return var / (mu * mu + 1e-10)


def _routing_kernel(x_ref, wg_ref, loss_ref, g2d_ref, dest_ref, texp_ref,
                    nt_ref, *, maxt):
    x = x_ref[...]
    wg = wg_ref[...]
    B = x.shape[0]
    ne = wg.shape[1]
    logits = lax.dot_general(
        x, wg, (((1,), (0,)), ((), ())), preferred_element_type=jnp.float32
    )
    cols = lax.broadcasted_iota(jnp.int32, logits.shape, 1)
    m1 = jnp.max(logits, axis=1, keepdims=True)
    i1 = jnp.min(jnp.where(logits == m1, cols, ne), axis=1, keepdims=True)
    masked = jnp.where(cols == i1, -jnp.inf, logits)
    m2 = jnp.max(masked, axis=1, keepdims=True)
    i2 = jnp.min(jnp.where(masked == m2, cols, ne), axis=1, keepdims=True)
    e2 = jnp.exp(m2 - m1)
    denom = 1.0 + e2
    g1 = 1.0 / denom
    g2 = e2 / denom

    oh1 = (cols == i1).astype(jnp.float32)
    oh2 = (cols == i2).astype(jnp.float32)
    gates = oh1 * g1 + jnp.where(g2 > 0, oh2 * g2, 0.0)
    importance = jnp.sum(gates, axis=0)
    load = jnp.sum((gates > 0).astype(jnp.float32), axis=0)
    loss_ref[...] = ((_cv_sq(importance) + _cv_sq(load)) * _LOSS_COEF)[
        None, None
    ]

    gcols = lax.broadcasted_iota(jnp.int32, g2d_ref.shape, 1)
    g2d_ref[...] = jnp.where(
        gcols == 0, g1, jnp.where(gcols == 1, g2, 0.0)
    )

    oh = jnp.concatenate([oh1, oh2], axis=0)
    c = oh
    step = 1
    while step < 2 * B:
        c = c + jnp.concatenate(
            [jnp.zeros((step, ne), jnp.float32), c[: 2 * B - step, :]], axis=0
        )
        step *= 2
    counts = c[2 * B - 1 : 2 * B, :]
    cnt_pad = jnp.ceil(counts / _T) * _T
    rl = lax.broadcasted_iota(jnp.int32, (ne, ne), 0)
    cl = lax.broadcasted_iota(jnp.int32, (ne, ne), 1)
    lower = (rl < cl).astype(jnp.float32)
    offs = lax.dot_general(
        cnt_pad, lower, (((1,), (0,)), ((), ())),
        preferred_element_type=jnp.float32,
    )
    ohs = jnp.concatenate([oh1, oh2], axis=0)
    dest = jnp.sum(ohs * (offs + c - 1.0), axis=1, keepdims=True)
    dest_ref[...] = dest.astype(jnp.int32)

    ends = offs + cnt_pad
    jt = lax.broadcasted_iota(jnp.int32, (maxt, 1), 0).astype(jnp.float32) * _T
    texp = jnp.sum((jt >= ends).astype(jnp.float32), axis=1, keepdims=True)
    texp_ref[...] = jnp.minimum(texp, float(ne - 1)).astype(jnp.int32)
    nt_ref[...] = (jnp.sum(cnt_pad) / _T).astype(jnp.int32)[None, None]


def _expert_kernel(texp_ref, nt_ref, xs_ref, W1_ref, b1_ref, W2_ref, b2_ref,
                   out_ref, oe_acc, sem, *, nhb, tt):
    hb = pl.program_id(0)
    j = pl.program_id(1)
    T = xs_ref.shape[0]

    @pl.when(j < nt_ref[0])
    def _():
        rows = pl.ds(j * T, T)
        h = lax.dot_general(
            xs_ref[...], W1_ref[0], (((1,), (0,)), ((), ())),
            preferred_element_type=jnp.float32,
        )
        h = jnp.maximum(h + b1_ref[0], 0.0)
        partial = lax.dot_general(
            h, W2_ref[0], (((1,), (0,)), ((), ())),
            preferred_element_type=jnp.float32,
        )

        @pl.when(hb == 0)
        def _():
            oe_acc[rows, :] = partial

        @pl.when(hb > 0)
        def _():
            oe_acc[rows, :] += partial

        @pl.when(hb == nhb - 1)
        def _():
            oe_acc[rows, :] = jnp.exp(oe_acc[rows, :] + b2_ref[0])
            copy = pltpu.make_async_copy(
                oe_acc.at[rows, :], out_ref.at[rows, :], sem
            )
            copy.start()
            copy.wait()


def _finalize_kernel(c_ref, g2d_ref, y_ref):
    c0 = c_ref[0]
    c1 = c_ref[1]
    g1 = g2d_ref[:, 0:1]
    g2 = g2d_ref[:, 1:2]
    acc = jnp.where(g1 > 0, g1 * c0, 0.0) + jnp.where(g2 > 0, g2 * c1, 0.0)
    y_ref[...] = jnp.log(jnp.where(acc == 0.0, jnp.float32(_EPS), acc))


def _sc_dispatch(x, dest, buf_rows, n_chunks):
    M = dest.shape[0]
    B, D = x.shape
    info = plsc.get_sparse_core_info()
    NW = info.num_cores * info.num_subcores
    per_w = buf_rows // NW
    ch = per_w // n_chunks

    @functools.partial(
        pl.kernel,
        out_type=jax.ShapeDtypeStruct((buf_rows, D), jnp.float32),
        mesh=plsc.VectorSubcoreMesh(core_axis_name="c", subcore_axis_name="s"),
        scratch_types=[
            pltpu.VMEM((M,), jnp.int32),
            pltpu.VMEM((buf_rows,), jnp.int32),
            pltpu.VMEM_SHARED((buf_rows,), jnp.int32),
            pltpu.VMEM((ch,), jnp.int32),
            pltpu.VMEM((ch, D), jnp.float32),
            pltpu.SemaphoreType.DMA,
        ],
        compiler_params=pltpu.CompilerParams(needs_layout_passes=False),
    )
    def k(x_hbm, dest_hbm, out_hbm, dest_v, tok_v, tok_s, idx_v, rows_v, sem):
        cid = lax.axis_index("c")
        sid = lax.axis_index("s")
        wid = sid * info.num_cores + cid

        @pl.when(sid == 0)
        def _():
            zeros = jnp.zeros((16,), jnp.int32)

            def init_body(i, _):
                tok_v[pl.ds(i * 16, 16)] = zeros
                return 0

            lax.fori_loop(0, buf_rows // 16, init_body, 0)
            pltpu.sync_copy(dest_hbm, dest_v)
            lane = lax.iota(jnp.int32, 16)

            def body(i, _):
                idx = dest_v[pl.ds(i * 16, 16)]
                tok = (lane + i * 16) & (B - 1)
                plsc.store_scatter(tok_v, [idx], tok)
                return 0

            lax.fori_loop(0, M // 16, body, 0)
            pltpu.sync_copy(tok_v, tok_s)

        plsc.subcore_barrier()
        base = wid * per_w
        for c in range(n_chunks):
            off = base + c * ch
            pltpu.sync_copy(tok_s.at[pl.ds(off, ch)], idx_v)
            pltpu.async_copy(x_hbm.at[idx_v], rows_v, sem).wait()
            pltpu.sync_copy(rows_v, out_hbm.at[pl.ds(off, ch)])

    return k(x, dest)


def _sc_gather_rows(table, idx, n_chunks):
    M = idx.shape[0]
    D = table.shape[1]
    info = plsc.get_sparse_core_info()
    NW = info.num_cores * info.num_subcores
    per_w = M // NW
    ch = per_w // n_chunks

    @functools.partial(
        pl.kernel,
        out_type=jax.ShapeDtypeStruct((M, D), jnp.float32),
        mesh=plsc.VectorSubcoreMesh(core_axis_name="c", subcore_axis_name="s"),
        scratch_types=[
            pltpu.VMEM((ch,), jnp.int32),
            pltpu.VMEM((ch, D), jnp.float32),
            pltpu.SemaphoreType.DMA,
        ],
    )
    def k(table_hbm, idx_hbm, out_hbm, idx_v, rows_v, sem):
        wid = lax.axis_index("s") * info.num_cores + lax.axis_index("c")
        base = wid * per_w
        for c in range(n_chunks):
            off = base + c * ch
            pltpu.sync_copy(idx_hbm.at[pl.ds(off, ch)], idx_v)
            pltpu.async_copy(table_hbm.at[idx_v], rows_v, sem).wait()
            pltpu.sync_copy(rows_v, out_hbm.at[pl.ds(off, ch)])

    return k(table, idx)


def kernel(x, w_gate, W1, b1, W2, b2):
    B, D = x.shape
    ne = W1.shape[0]
    H = W1.shape[2]
    O = W2.shape[2]
    hbw = min(_HB, H)
    nhb = H // hbw
    maxt = -(-2 * B // _T) + ne - 1
    maxt += (-maxt) % 8
    buf = maxt * _T

    loss2d, g2d, dest2d, texp2d, nt2d = pl.pallas_call(
        functools.partial(_routing_kernel, maxt=maxt),
        out_shape=(
            jax.ShapeDtypeStruct((1, 1), jnp.float32),
            jax.ShapeDtypeStruct((B, 128), jnp.float32),
            jax.ShapeDtypeStruct((2 * B, 1), jnp.int32),
            jax.ShapeDtypeStruct((maxt, 1), jnp.int32),
            jax.ShapeDtypeStruct((1, 1), jnp.int32),
        ),
    )(x, w_gate)

    dest = dest2d.reshape(2 * B)
    xs = _sc_dispatch(x, dest, buf, 4)

    b1r = b1.reshape(ne, 1, H)
    b2r = b2.reshape(ne, 1, O)
    texp = texp2d.reshape(maxt)
    nt = nt2d.reshape(1)

    contrib = pl.pallas_call(
        functools.partial(_expert_kernel, nhb=nhb, tt=maxt),
        grid_spec=pltpu.PrefetchScalarGridSpec(
            num_scalar_prefetch=2,
            grid=(nhb, maxt),
            in_specs=[
                pl.BlockSpec((_T, D), lambda hb, j, texp, nt: (j, 0)),
                pl.BlockSpec((1, D, hbw), lambda hb, j, texp, nt: (texp[j], 0, hb)),
                pl.BlockSpec((1, 1, hbw), lambda hb, j, texp, nt: (texp[j], 0, hb)),
                pl.BlockSpec((1, hbw, O), lambda hb, j, texp, nt: (texp[j], hb, 0)),
                pl.BlockSpec((1, 1, O), lambda hb, j, texp, nt: (texp[j], 0, 0)),
            ],
            out_specs=pl.BlockSpec(memory_space=pl.ANY),
            scratch_shapes=[
                pltpu.VMEM((buf, O), jnp.float32),
                pltpu.SemaphoreType.DMA,
            ],
        ),
        out_shape=jax.ShapeDtypeStruct((buf, O), jnp.float32),
        compiler_params=pltpu.CompilerParams(
            dimension_semantics=("arbitrary", "arbitrary"),
            vmem_limit_bytes=100 * 1024 * 1024,
        ),
    )(texp, nt, xs, W1, b1r, W2, b2r)

    crows = _sc_gather_rows(contrib, dest, 2).reshape(2, B, O)

    y = pl.pallas_call(
        _finalize_kernel,
        grid=(B // _T,),
        in_specs=[
            pl.BlockSpec((2, _T, O), lambda t: (0, t, 0)),
            pl.BlockSpec((_T, 128), lambda t: (t, 0)),
        ],
        out_specs=pl.BlockSpec((_T, O), lambda t: (t, 0)),
        out_shape=jax.ShapeDtypeStruct((B, O), jnp.float32),
    )(crows, g2d)

    return y, loss2d[0, 0]


# speedup vs baseline: 1.6495x; 1.4025x over previous
import functools

import jax
import jax.numpy as jnp
import numpy as np
from jax import lax
from jax.experimental import pallas as pl
from jax.experimental.pallas import tpu as pltpu
from jax.experimental.pallas import tpu_sc as plsc

_LOSS_COEF = 1e-2
_EPS = float(np.finfo(float).eps)

_T = 256
_HB = 1024


def _cv_sq(v):
    n = v.shape[0]
    mu = jnp.mean(v)
    var = jnp.sum((v - mu) ** 2) / (n - 1)
    return var / (mu * mu + 1e-10)


def _routing_kernel(x_ref, wg_ref, loss_ref, g2d_ref, dest_ref, texp_ref,
                    nt_ref, *, maxt):
    x = x_ref[...]
    wg = wg_ref[...]
    B = x.shape[0]
    ne = wg.shape[1]
    logits = lax.dot_general(
        x, wg, (((1,), (0,)), ((), ())), preferred_element_type=jnp.float32
    )
    cols = lax.broadcasted_iota(jnp.int32, logits.shape, 1)
    m1 = jnp.max(logits, axis=1, keepdims=True)
    i1 = jnp.min(jnp.where(logits == m1, cols, ne), axis=1, keepdims=True)
    masked = jnp.where(cols == i1, -jnp.inf, logits)
    m2 = jnp.max(masked, axis=1, keepdims=True)
    i2 = jnp.min(jnp.where(masked == m2, cols, ne), axis=1, keepdims=True)
    e2 = jnp.exp(m2 - m1)
    denom = 1.0 + e2
    g1 = 1.0 / denom
    g2 = e2 / denom

    oh1 = (cols == i1).astype(jnp.float32)
    oh2 = (cols == i2).astype(jnp.float32)
    gates = oh1 * g1 + jnp.where(g2 > 0, oh2 * g2, 0.0)
    importance = jnp.sum(gates, axis=0)
    load = jnp.sum((gates > 0).astype(jnp.float32), axis=0)
    loss_ref[...] = ((_cv_sq(importance) + _cv_sq(load)) * _LOSS_COEF)[
        None, None
    ]

    gcols = lax.broadcasted_iota(jnp.int32, g2d_ref.shape, 1)
    g2d_ref[...] = jnp.where(
        gcols == 0, g1, jnp.where(gcols == 1, g2, 0.0)
    )

    oh = jnp.concatenate([oh1, oh2], axis=0)
    c = oh
    step = 1
    while step < 2 * B:
        c = c + jnp.concatenate(
            [jnp.zeros((step, ne), jnp.float32), c[: 2 * B - step, :]], axis=0
        )
        step *= 2
    counts = c[2 * B - 1 : 2 * B, :]
    cnt_pad = jnp.ceil(counts / _T) * _T
    rl = lax.broadcasted_iota(jnp.int32, (ne, ne), 0)
    cl = lax.broadcasted_iota(jnp.int32, (ne, ne), 1)
    lower = (rl < cl).astype(jnp.float32)
    offs = lax.dot_general(
        cnt_pad, lower, (((1,), (0,)), ((), ())),
        preferred_element_type=jnp.float32,
    )
    ohs = jnp.concatenate([oh1, oh2], axis=0)
    dest = jnp.sum(ohs * (offs + c - 1.0), axis=1, keepdims=True)
    dest_ref[...] = dest.astype(jnp.int32)

    ends = offs + cnt_pad
    jt = lax.broadcasted_iota(jnp.int32, (maxt, 1), 0).astype(jnp.float32) * _T
    texp = jnp.sum((jt >= ends).astype(jnp.float32), axis=1, keepdims=True)
    texp_ref[...] = jnp.minimum(texp, float(ne - 1)).astype(jnp.int32)
    nt_ref[...] = (jnp.sum(cnt_pad) / _T).astype(jnp.int32)[None, None]


def _expert_kernel(texp_ref, nt_ref, xs_ref, W1_ref, b1_ref, W2_ref, b2_ref,
                   out_ref, oe_acc, sem, *, nhb, tt):
    hb = pl.program_id(0)
    j = pl.program_id(1)
    T = xs_ref.shape[0]

    @pl.when(j < nt_ref[0])
    def _():
        rows = pl.ds(j * T, T)
        h = lax.dot_general(
            xs_ref[...], W1_ref[0], (((1,), (0,)), ((), ())),
            preferred_element_type=jnp.float32,
        )
        h = jnp.maximum(h + b1_ref[0], 0.0)
        partial = lax.dot_general(
            h, W2_ref[0], (((1,), (0,)), ((), ())),
            preferred_element_type=jnp.float32,
        )

        @pl.when(hb == 0)
        def _():
            oe_acc[rows, :] = partial

        @pl.when(hb > 0)
        def _():
            oe_acc[rows, :] += partial

        @pl.when(hb == nhb - 1)
        def _():
            oe_acc[rows, :] = jnp.exp(oe_acc[rows, :] + b2_ref[0])
            copy = pltpu.make_async_copy(
                oe_acc.at[rows, :], out_ref.at[rows, :], sem
            )
            copy.start()
            copy.wait()


def _finalize_kernel(c_ref, g2d_ref, y_ref):
    c0 = c_ref[0]
    c1 = c_ref[1]
    g1 = g2d_ref[:, 0:1]
    g2 = g2d_ref[:, 1:2]
    acc = jnp.where(g1 > 0, g1 * c0, 0.0) + jnp.where(g2 > 0, g2 * c1, 0.0)
    y_ref[...] = jnp.log(jnp.where(acc == 0.0, jnp.float32(_EPS), acc))


def _sc_dispatch(x, dest, buf_rows):
    B, D = x.shape
    info = plsc.get_sparse_core_info()
    NW = info.num_cores * info.num_subcores
    tpw = B // NW

    @functools.partial(
        pl.kernel,
        out_type=jax.ShapeDtypeStruct((buf_rows, D), jnp.float32),
        mesh=plsc.VectorSubcoreMesh(core_axis_name="c", subcore_axis_name="s"),
        scratch_types=[
            pltpu.VMEM((tpw,), jnp.int32),
            pltpu.VMEM((tpw,), jnp.int32),
            pltpu.VMEM((tpw, D), jnp.float32),
            pltpu.SemaphoreType.DMA,
        ],
        compiler_params=pltpu.CompilerParams(needs_layout_passes=False),
    )
    def k(x_hbm, dest_hbm, out_hbm, idx0, idx1, rows_v, sem):
        wid = lax.axis_index("s") * info.num_cores + lax.axis_index("c")
        tb = wid * tpw
        pltpu.sync_copy(x_hbm.at[pl.ds(tb, tpw)], rows_v)
        pltpu.sync_copy(dest_hbm.at[pl.ds(tb, tpw)], idx0)
        pltpu.sync_copy(dest_hbm.at[pl.ds(B + tb, tpw)], idx1)
        pltpu.async_copy(rows_v, out_hbm.at[idx0], sem).wait()
        pltpu.async_copy(rows_v, out_hbm.at[idx1], sem).wait()

    return k(x, dest)


def _sc_gather_rows(table, idx, n_chunks):
    M = idx.shape[0]
    D = table.shape[1]
    info = plsc.get_sparse_core_info()
    NW = info.num_cores * info.num_subcores
    per_w = M // NW
    ch = per_w // n_chunks

    @functools.partial(
        pl.kernel,
        out_type=jax.ShapeDtypeStruct((M, D), jnp.float32),
        mesh=plsc.VectorSubcoreMesh(core_axis_name="c", subcore_axis_name="s"),
        scratch_types=[
            pltpu.VMEM((ch,), jnp.int32),
            pltpu.VMEM((ch, D), jnp.float32),
            pltpu.SemaphoreType.DMA,
        ],
    )
    def k(table_hbm, idx_hbm, out_hbm, idx_v, rows_v, sem):
        wid = lax.axis_index("s") * info.num_cores + lax.axis_index("c")
        base = wid * per_w
        for c in range(n_chunks):
            off = base + c * ch
            pltpu.sync_copy(idx_hbm.at[pl.ds(off, ch)], idx_v)
            pltpu.async_copy(table_hbm.at[idx_v], rows_v, sem).wait()
            pltpu.sync_copy(rows_v, out_hbm.at[pl.ds(off, ch)])

    return k(table, idx)


def kernel(x, w_gate, W1, b1, W2, b2):
    B, D = x.shape
    ne = W1.shape[0]
    H = W1.shape[2]
    O = W2.shape[2]
    hbw = min(_HB, H)
    nhb = H // hbw
    maxt = -(-2 * B // _T) + ne - 1
    maxt += (-maxt) % 8
    buf = maxt * _T

    loss2d, g2d, dest2d, texp2d, nt2d = pl.pallas_call(
        functools.partial(_routing_kernel, maxt=maxt),
        out_shape=(
            jax.ShapeDtypeStruct((1, 1), jnp.float32),
            jax.ShapeDtypeStruct((B, 128), jnp.float32),
            jax.ShapeDtypeStruct((2 * B, 1), jnp.int32),
            jax.ShapeDtypeStruct((maxt, 1), jnp.int32),
            jax.ShapeDtypeStruct((1, 1), jnp.int32),
        ),
    )(x, w_gate)

    dest = dest2d.reshape(2 * B)
    xs = _sc_dispatch(x, dest, buf)

    b1r = b1.reshape(ne, 1, H)
    b2r = b2.reshape(ne, 1, O)
    texp = texp2d.reshape(maxt)
    nt = nt2d.reshape(1)

    contrib = pl.pallas_call(
        functools.partial(_expert_kernel, nhb=nhb, tt=maxt),
        grid_spec=pltpu.PrefetchScalarGridSpec(
            num_scalar_prefetch=2,
            grid=(nhb, maxt),
            in_specs=[
                pl.BlockSpec((_T, D), lambda hb, j, texp, nt: (j, 0)),
                pl.BlockSpec((1, D, hbw), lambda hb, j, texp, nt: (texp[j], 0, hb)),
                pl.BlockSpec((1, 1, hbw), lambda hb, j, texp, nt: (texp[j], 0, hb)),
                pl.BlockSpec((1, hbw, O), lambda hb, j, texp, nt: (texp[j], hb, 0)),
                pl.BlockSpec((1, 1, O), lambda hb, j, texp, nt: (texp[j], 0, 0)),
            ],
            out_specs=pl.BlockSpec(memory_space=pl.ANY),
            scratch_shapes=[
                pltpu.VMEM((buf, O), jnp.float32),
                pltpu.SemaphoreType.DMA,
            ],
        ),
        out_shape=jax.ShapeDtypeStruct((buf, O), jnp.float32),
        compiler_params=pltpu.CompilerParams(
            dimension_semantics=("arbitrary", "arbitrary"),
            vmem_limit_bytes=100 * 1024 * 1024,
        ),
    )(texp, nt, xs, W1, b1r, W2, b2r)

    crows = _sc_gather_rows(contrib, dest, 2).reshape(2, B, O)

    y = pl.pallas_call(
        _finalize_kernel,
        grid=(B // _T,),
        in_specs=[
            pl.BlockSpec((2, _T, O), lambda t: (0, t, 0)),
            pl.BlockSpec((_T, 128), lambda t: (t, 0)),
        ],
        out_specs=pl.BlockSpec((_T, O), lambda t: (t, 0)),
        out_shape=jax.ShapeDtypeStruct((B, O), jnp.float32),
    )(crows, g2d)

    return y, loss2d[0, 0]


# speedup vs baseline: 1.8390x; 1.1149x over previous
import functools

import jax
import jax.numpy as jnp
import numpy as np
from jax import lax
from jax.experimental import pallas as pl
from jax.experimental.pallas import tpu as pltpu
from jax.experimental.pallas import tpu_sc as plsc

_LOSS_COEF = 1e-2
_EPS = float(np.finfo(float).eps)

_T = 512
_HB = 1024


def _cv_sq(v):
    n = v.shape[0]
    mu = jnp.mean(v)
    var = jnp.sum((v - mu) ** 2) / (n - 1)
    return var / (mu * mu + 1e-10)


def _routing_kernel(x_ref, wg_ref, loss_ref, g2d_ref, dest_ref, texp_ref,
                    nt_ref, *, maxt):
    x = x_ref[...]
    wg = wg_ref[...]
    B = x.shape[0]
    ne = wg.shape[1]
    logits = lax.dot_general(
        x, wg, (((1,), (0,)), ((), ())), preferred_element_type=jnp.float32
    )
    cols = lax.broadcasted_iota(jnp.int32, logits.shape, 1)
    m1 = jnp.max(logits, axis=1, keepdims=True)
    i1 = jnp.min(jnp.where(logits == m1, cols, ne), axis=1, keepdims=True)
    masked = jnp.where(cols == i1, -jnp.inf, logits)
    m2 = jnp.max(masked, axis=1, keepdims=True)
    i2 = jnp.min(jnp.where(masked == m2, cols, ne), axis=1, keepdims=True)
    e2 = jnp.exp(m2 - m1)
    denom = 1.0 + e2
    g1 = 1.0 / denom
    g2 = e2 / denom

    oh1 = (cols == i1).astype(jnp.float32)
    oh2 = (cols == i2).astype(jnp.float32)
    gates = oh1 * g1 + jnp.where(g2 > 0, oh2 * g2, 0.0)
    importance = jnp.sum(gates, axis=0)
    load = jnp.sum((gates > 0).astype(jnp.float32), axis=0)
    loss_ref[...] = ((_cv_sq(importance) + _cv_sq(load)) * _LOSS_COEF)[
        None, None
    ]

    gcols = lax.broadcasted_iota(jnp.int32, g2d_ref.shape, 1)
    g2d_ref[...] = jnp.where(
        gcols == 0, g1, jnp.where(gcols == 1, g2, 0.0)
    )

    oh = jnp.concatenate([oh1, oh2], axis=0)
    c = oh
    step = 1
    while step < 2 * B:
        c = c + jnp.concatenate(
            [jnp.zeros((step, ne), jnp.float32), c[: 2 * B - step, :]], axis=0
        )
        step *= 2
    counts = c[2 * B - 1 : 2 * B, :]
    cnt_pad = jnp.ceil(counts / _T) * _T
    rl = lax.broadcasted_iota(jnp.int32, (ne, ne), 0)
    cl = lax.broadcasted_iota(jnp.int32, (ne, ne), 1)
    lower = (rl < cl).astype(jnp.float32)
    offs = lax.dot_general(
        cnt_pad, lower, (((1,), (0,)), ((), ())),
        preferred_element_type=jnp.float32,
    )
    ohs = jnp.concatenate([oh1, oh2], axis=0)
    dest = jnp.sum(ohs * (offs + c - 1.0), axis=1, keepdims=True)
    dest_ref[...] = dest.astype(jnp.int32)

    ends = offs + cnt_pad
    jt = lax.broadcasted_iota(jnp.int32, (maxt, 1), 0).astype(jnp.float32) * _T
    texp = jnp.sum((jt >= ends).astype(jnp.float32), axis=1, keepdims=True)
    texp_ref[...] = jnp.minimum(texp, float(ne - 1)).astype(jnp.int32)
    nt_ref[...] = (jnp.sum(cnt_pad) / _T).astype(jnp.int32)[None, None]


def _expert_kernel(texp_ref, nt_ref, xs_ref, W1_ref, b1_ref, W2_ref, b2_ref,
                   out_ref, oe_acc, sem, *, nhb, tt):
    hb = pl.program_id(0)
    j = pl.program_id(1)
    T = xs_ref.shape[0]

    @pl.when(j < nt_ref[0])
    def _():
        rows = pl.ds(j * T, T)
        h = lax.dot_general(
            xs_ref[...], W1_ref[0], (((1,), (0,)), ((), ())),
            preferred_element_type=jnp.float32,
        )
        h = jnp.maximum(h + b1_ref[0], 0.0)
        partial = lax.dot_general(
            h, W2_ref[0], (((1,), (0,)), ((), ())),
            preferred_element_type=jnp.float32,
        )

        @pl.when(hb == 0)
        def _():
            oe_acc[rows, :] = partial

        @pl.when(hb > 0)
        def _():
            oe_acc[rows, :] += partial

        @pl.when(hb == nhb - 1)
        def _():
            oe_acc[rows, :] = jnp.exp(oe_acc[rows, :] + b2_ref[0])
            copy = pltpu.make_async_copy(
                oe_acc.at[rows, :], out_ref.at[rows, :], sem
            )
            copy.start()
            copy.wait()


def _finalize_kernel(c_ref, g2d_ref, y_ref):
    c0 = c_ref[0]
    c1 = c_ref[1]
    g1 = g2d_ref[:, 0:1]
    g2 = g2d_ref[:, 1:2]
    acc = jnp.where(g1 > 0, g1 * c0, 0.0) + jnp.where(g2 > 0, g2 * c1, 0.0)
    y_ref[...] = jnp.log(jnp.where(acc == 0.0, jnp.float32(_EPS), acc))


def _sc_dispatch(x, dest, buf_rows):
    B, D = x.shape
    info = plsc.get_sparse_core_info()
    NW = info.num_cores * info.num_subcores
    tpw = B // NW

    @functools.partial(
        pl.kernel,
        out_type=jax.ShapeDtypeStruct((buf_rows, D), jnp.float32),
        mesh=plsc.VectorSubcoreMesh(core_axis_name="c", subcore_axis_name="s"),
        scratch_types=[
            pltpu.VMEM((tpw,), jnp.int32),
            pltpu.VMEM((tpw,), jnp.int32),
            pltpu.VMEM((tpw, D), jnp.float32),
            pltpu.SemaphoreType.DMA,
        ],
        compiler_params=pltpu.CompilerParams(needs_layout_passes=False),
    )
    def k(x_hbm, dest_hbm, out_hbm, idx0, idx1, rows_v, sem):
        wid = lax.axis_index("s") * info.num_cores + lax.axis_index("c")
        tb = wid * tpw
        pltpu.sync_copy(x_hbm.at[pl.ds(tb, tpw)], rows_v)
        pltpu.sync_copy(dest_hbm.at[pl.ds(tb, tpw)], idx0)
        pltpu.sync_copy(dest_hbm.at[pl.ds(B + tb, tpw)], idx1)
        pltpu.async_copy(rows_v, out_hbm.at[idx0], sem).wait()
        pltpu.async_copy(rows_v, out_hbm.at[idx1], sem).wait()

    return k(x, dest)


def _sc_gather_rows(table, idx, n_chunks):
    M = idx.shape[0]
    D = table.shape[1]
    info = plsc.get_sparse_core_info()
    NW = info.num_cores * info.num_subcores
    per_w = M // NW
    ch = per_w // n_chunks

    @functools.partial(
        pl.kernel,
        out_type=jax.ShapeDtypeStruct((M, D), jnp.float32),
        mesh=plsc.VectorSubcoreMesh(core_axis_name="c", subcore_axis_name="s"),
        scratch_types=[
            pltpu.VMEM((ch,), jnp.int32),
            pltpu.VMEM((ch, D), jnp.float32),
            pltpu.SemaphoreType.DMA,
        ],
    )
    def k(table_hbm, idx_hbm, out_hbm, idx_v, rows_v, sem):
        wid = lax.axis_index("s") * info.num_cores + lax.axis_index("c")
        base = wid * per_w
        for c in range(n_chunks):
            off = base + c * ch
            pltpu.sync_copy(idx_hbm.at[pl.ds(off, ch)], idx_v)
            pltpu.async_copy(table_hbm.at[idx_v], rows_v, sem).wait()
            pltpu.sync_copy(rows_v, out_hbm.at[pl.ds(off, ch)])

    return k(table, idx)


def kernel(x, w_gate, W1, b1, W2, b2):
    B, D = x.shape
    ne = W1.shape[0]
    H = W1.shape[2]
    O = W2.shape[2]
    hbw = min(_HB, H)
    nhb = H // hbw
    maxt = -(-2 * B // _T) + ne - 1
    maxt += (-maxt) % 8
    buf = maxt * _T

    loss2d, g2d, dest2d, texp2d, nt2d = pl.pallas_call(
        functools.partial(_routing_kernel, maxt=maxt),
        out_shape=(
            jax.ShapeDtypeStruct((1, 1), jnp.float32),
            jax.ShapeDtypeStruct((B, 128), jnp.float32),
            jax.ShapeDtypeStruct((2 * B, 1), jnp.int32),
            jax.ShapeDtypeStruct((maxt, 1), jnp.int32),
            jax.ShapeDtypeStruct((1, 1), jnp.int32),
        ),
    )(x, w_gate)

    dest = dest2d.reshape(2 * B)
    xs = _sc_dispatch(x, dest, buf)

    b1r = b1.reshape(ne, 1, H)
    b2r = b2.reshape(ne, 1, O)
    texp = texp2d.reshape(maxt)
    nt = nt2d.reshape(1)

    contrib = pl.pallas_call(
        functools.partial(_expert_kernel, nhb=nhb, tt=maxt),
        grid_spec=pltpu.PrefetchScalarGridSpec(
            num_scalar_prefetch=2,
            grid=(nhb, maxt),
            in_specs=[
                pl.BlockSpec((_T, D), lambda hb, j, texp, nt: (j, 0)),
                pl.BlockSpec((1, D, hbw), lambda hb, j, texp, nt: (texp[j], 0, hb)),
                pl.BlockSpec((1, 1, hbw), lambda hb, j, texp, nt: (texp[j], 0, hb)),
                pl.BlockSpec((1, hbw, O), lambda hb, j, texp, nt: (texp[j], hb, 0)),
                pl.BlockSpec((1, 1, O), lambda hb, j, texp, nt: (texp[j], 0, 0)),
            ],
            out_specs=pl.BlockSpec(memory_space=pl.ANY),
            scratch_shapes=[
                pltpu.VMEM((buf, O), jnp.float32),
                pltpu.SemaphoreType.DMA,
            ],
        ),
        out_shape=jax.ShapeDtypeStruct((buf, O), jnp.float32),
        compiler_params=pltpu.CompilerParams(
            dimension_semantics=("arbitrary", "arbitrary"),
            vmem_limit_bytes=100 * 1024 * 1024,
        ),
    )(texp, nt, xs, W1, b1r, W2, b2r)

    crows = _sc_gather_rows(contrib, dest, 2).reshape(2, B, O)

    y = pl.pallas_call(
        _finalize_kernel,
        grid=(B // _T,),
        in_specs=[
            pl.BlockSpec((2, _T, O), lambda t: (0, t, 0)),
            pl.BlockSpec((_T, 128), lambda t: (t, 0)),
        ],
        out_specs=pl.BlockSpec((_T, O), lambda t: (t, 0)),
        out_shape=jax.ShapeDtypeStruct((B, O), jnp.float32),
    )(crows, g2d)

    return y, loss2d[0, 0]
